# initial kernel scaffold (unmeasured)
import jax
import jax.numpy as jnp
from jax import lax
from jax.experimental import pallas as pl
from jax.experimental.pallas import tpu as pltpu

N_DEV = 4


def kernel(x, w_mat, scale_x, scale_w):
    m_per, k = x.shape
    n = w_mat.shape[1]
    n_per = n // N_DEV

    my = lax.axis_index("i")
    x8 = x.astype(jnp.float8_e5m2)
    w_loc = lax.dynamic_slice(w_mat, (0, my * n_per), (k, n_per)).astype(
        jnp.float8_e5m2
    )
    s = (scale_x * scale_w).astype(jnp.float32)

    def body(x_ref, w_ref, s_ref, out_ref, comm_ref, acc_ref, send_sems,
             recv_sems, copy_sem):
        my_pos = lax.axis_index("i")
        left = (my_pos - 1) % N_DEV
        right = (my_pos + 1) % N_DEV

        barrier_sem = pltpu.get_barrier_semaphore()
        for nbr in [left, right]:
            pl.semaphore_signal(
                barrier_sem, inc=1,
                device_id=(nbr,), device_id_type=pl.DeviceIdType.MESH,
            )
        pl.semaphore_wait(barrier_sem, 2)

        scale = s_ref[0]

        comm_ref[0] = x_ref[...]
        acc_ref[...] = (
            jnp.dot(x_ref[...], w_ref[...], preferred_element_type=jnp.float32)
            * scale
        )
        cp = pltpu.make_async_copy(
            acc_ref, out_ref.at[pl.ds(my_pos * m_per, m_per), :], copy_sem
        )
        cp.start()
        cp.wait()

        for h in range(N_DEV - 1):
            send_slot = h % 2
            recv_slot = (h + 1) % 2
            rdma = pltpu.make_async_remote_copy(
                src_ref=comm_ref.at[send_slot],
                dst_ref=comm_ref.at[recv_slot],
                send_sem=send_sems.at[send_slot],
                recv_sem=recv_sems.at[recv_slot],
                device_id=(right,),
                device_id_type=pl.DeviceIdType.MESH,
            )
            rdma.start()
            rdma.wait()

            origin = (my_pos - h - 1) % N_DEV
            acc_ref[...] = (
                jnp.dot(comm_ref[recv_slot], w_ref[...],
                        preferred_element_type=jnp.float32)
                * scale
            )
            cp = pltpu.make_async_copy(
                acc_ref, out_ref.at[pl.ds(origin * m_per, m_per), :], copy_sem
            )
            cp.start()
            cp.wait()

    return pl.pallas_call(
        body,
        out_shape=jax.ShapeDtypeStruct((N_DEV * m_per, n_per), jnp.float32),
        in_specs=[
            pl.BlockSpec(memory_space=pltpu.VMEM),
            pl.BlockSpec(memory_space=pltpu.VMEM),
            pl.BlockSpec(memory_space=pltpu.SMEM),
        ],
        out_specs=pl.BlockSpec(memory_space=pltpu.ANY),
        scratch_shapes=[
            pltpu.VMEM((2, m_per, k), jnp.float8_e5m2),
            pltpu.VMEM((m_per, n_per), jnp.float32),
            pltpu.SemaphoreType.DMA((2,)),
            pltpu.SemaphoreType.DMA((2,)),
            pltpu.SemaphoreType.DMA,
        ],
        compiler_params=pltpu.CompilerParams(collective_id=0),
    )(x8, w_loc, s)


# baseline (device time: 238582 ns/iter reference)
import jax
import jax.numpy as jnp
from jax import lax
from jax.experimental import pallas as pl
from jax.experimental.pallas import tpu as pltpu

N_DEV = 4


def kernel(x, w_mat, scale_x, scale_w):
    m_per, k = x.shape
    n = w_mat.shape[1]
    n_per = n // N_DEV

    my = lax.axis_index("i")
    x8 = x.astype(jnp.float8_e5m2)
    w_loc = lax.dynamic_slice(w_mat, (0, my * n_per), (k, n_per)).astype(
        jnp.float8_e5m2
    )
    s = (scale_x * scale_w).astype(jnp.float32)

    def body(x_ref, w_ref, s_ref, out_ref, comm_ref, acc_ref, send_sems,
             recv_sems, copy_sem):
        my_pos = lax.axis_index("i")
        left = (my_pos - 1) % N_DEV
        right = (my_pos + 1) % N_DEV

        barrier_sem = pltpu.get_barrier_semaphore()
        for nbr in [left, right]:
            pl.semaphore_signal(
                barrier_sem, inc=1,
                device_id=(nbr,), device_id_type=pl.DeviceIdType.MESH,
            )
        pl.semaphore_wait(barrier_sem, 2)

        scale = s_ref[0]

        comm_ref[0] = x_ref[...]
        acc_ref[...] = (
            jnp.dot(x_ref[...], w_ref[...], preferred_element_type=jnp.float32)
            * scale
        )
        cp = pltpu.make_async_copy(
            acc_ref, out_ref.at[pl.ds(my_pos * m_per, m_per), :], copy_sem
        )
        cp.start()
        cp.wait()

        for h in range(N_DEV - 1):
            send_slot = h % 2
            recv_slot = (h + 1) % 2
            rdma = pltpu.make_async_remote_copy(
                src_ref=comm_ref.at[send_slot],
                dst_ref=comm_ref.at[recv_slot],
                send_sem=send_sems.at[send_slot],
                recv_sem=recv_sems.at[recv_slot],
                device_id=(right,),
                device_id_type=pl.DeviceIdType.MESH,
            )
            rdma.start()
            rdma.wait()

            origin = (my_pos - h - 1) % N_DEV
            acc_ref[...] = (
                jnp.dot(comm_ref[recv_slot], w_ref[...],
                        preferred_element_type=jnp.float32)
                * scale
            )
            cp = pltpu.make_async_copy(
                acc_ref, out_ref.at[pl.ds(origin * m_per, m_per), :], copy_sem
            )
            cp.start()
            cp.wait()

    return pl.pallas_call(
        body,
        out_shape=jax.ShapeDtypeStruct((N_DEV * m_per, n_per), jnp.float32),
        in_specs=[
            pl.BlockSpec(memory_space=pltpu.VMEM),
            pl.BlockSpec(memory_space=pltpu.VMEM),
            pl.BlockSpec(memory_space=pltpu.SMEM),
        ],
        out_specs=pl.BlockSpec(memory_space=pl.ANY),
        scratch_shapes=[
            pltpu.VMEM((2, m_per, k), jnp.float8_e5m2),
            pltpu.VMEM((m_per, n_per), jnp.float32),
            pltpu.SemaphoreType.DMA((2,)),
            pltpu.SemaphoreType.DMA((2,)),
            pltpu.SemaphoreType.DMA,
        ],
        compiler_params=pltpu.CompilerParams(collective_id=0),
    )(x8, w_loc, s)


# device time: 132489 ns/iter; 1.8008x vs baseline; 1.8008x over previous
import jax
import jax.numpy as jnp
from jax import lax
from jax.experimental import pallas as pl
from jax.experimental.pallas import tpu as pltpu

N_DEV = 4


def kernel(x, w_mat, scale_x, scale_w):
    m_per, k = x.shape
    n = w_mat.shape[1]
    n_per = n // N_DEV
    h_per = m_per // 2

    my = lax.axis_index("i")
    x8 = x.astype(jnp.float8_e5m2)
    w_loc = lax.dynamic_slice(w_mat, (0, my * n_per), (k, n_per)).astype(
        jnp.float8_e5m2
    )
    s = (scale_x * scale_w).astype(jnp.float32)

    def body(x_ref, w_ref, s_ref, out_ref, cw_ref, ccw_ref, acc_ref,
             cw_send, cw_recv, ccw_send, ccw_recv, copy_sems):
        my_pos = lax.axis_index("i")
        left = (my_pos - 1) % N_DEV
        right = (my_pos + 1) % N_DEV

        barrier_sem = pltpu.get_barrier_semaphore()
        for nbr in [left, right]:
            pl.semaphore_signal(
                barrier_sem, inc=1,
                device_id=(nbr,), device_id_type=pl.DeviceIdType.MESH,
            )
        pl.semaphore_wait(barrier_sem, 2)

        scale = s_ref[0]

        cw_ref[0] = x_ref[:h_per, :]
        ccw_ref[0] = x_ref[h_per:, :]

        pending = []

        def compute_store(src, row0, sem_slot):
            acc_ref[sem_slot] = (
                jnp.dot(src, w_ref[...], preferred_element_type=jnp.float32)
                * scale
            )
            cp = pltpu.make_async_copy(
                acc_ref.at[sem_slot],
                out_ref.at[pl.ds(row0, h_per), :],
                copy_sems.at[sem_slot],
            )
            cp.start()
            pending.append(cp)

        for hop in range(N_DEV - 1):
            s_slot = hop % 2
            r_slot = (hop + 1) % 2
            cw_rdma = pltpu.make_async_remote_copy(
                src_ref=cw_ref.at[s_slot],
                dst_ref=cw_ref.at[r_slot],
                send_sem=cw_send.at[s_slot],
                recv_sem=cw_recv.at[r_slot],
                device_id=(right,),
                device_id_type=pl.DeviceIdType.MESH,
            )
            ccw_rdma = pltpu.make_async_remote_copy(
                src_ref=ccw_ref.at[s_slot],
                dst_ref=ccw_ref.at[r_slot],
                send_sem=ccw_send.at[s_slot],
                recv_sem=ccw_recv.at[r_slot],
                device_id=(left,),
                device_id_type=pl.DeviceIdType.MESH,
            )
            cw_rdma.start()
            ccw_rdma.start()

            while pending:
                pending.pop(0).wait()

            if hop == 0:
                compute_store(x_ref[:h_per, :], my_pos * m_per, 0)
                compute_store(x_ref[h_per:, :], my_pos * m_per + h_per, 1)
            else:
                o_cw = (my_pos - hop) % N_DEV
                o_ccw = (my_pos + hop) % N_DEV
                compute_store(cw_ref[s_slot], o_cw * m_per, 0)
                compute_store(ccw_ref[s_slot], o_ccw * m_per + h_per, 1)

            cw_rdma.wait()
            ccw_rdma.wait()

        while pending:
            pending.pop(0).wait()
        o_cw = (my_pos - (N_DEV - 1)) % N_DEV
        o_ccw = (my_pos + (N_DEV - 1)) % N_DEV
        compute_store(cw_ref[1], o_cw * m_per, 0)
        compute_store(ccw_ref[1], o_ccw * m_per + h_per, 1)
        while pending:
            pending.pop(0).wait()

    return pl.pallas_call(
        body,
        out_shape=jax.ShapeDtypeStruct((N_DEV * m_per, n_per), jnp.float32),
        in_specs=[
            pl.BlockSpec(memory_space=pltpu.VMEM),
            pl.BlockSpec(memory_space=pltpu.VMEM),
            pl.BlockSpec(memory_space=pltpu.SMEM),
        ],
        out_specs=pl.BlockSpec(memory_space=pl.ANY),
        scratch_shapes=[
            pltpu.VMEM((2, h_per, k), jnp.float8_e5m2),
            pltpu.VMEM((2, h_per, k), jnp.float8_e5m2),
            pltpu.VMEM((2, h_per, n_per), jnp.float32),
            pltpu.SemaphoreType.DMA((2,)),
            pltpu.SemaphoreType.DMA((2,)),
            pltpu.SemaphoreType.DMA((2,)),
            pltpu.SemaphoreType.DMA((2,)),
            pltpu.SemaphoreType.DMA((2,)),
        ],
        compiler_params=pltpu.CompilerParams(collective_id=0),
    )(x8, w_loc, s)


# device time: 116217 ns/iter; 2.0529x vs baseline; 1.1400x over previous
import jax
import jax.numpy as jnp
from jax import lax
from jax.experimental import pallas as pl
from jax.experimental.pallas import tpu as pltpu

N_DEV = 4
W_CHUNKS = 8


def kernel(x, w_mat, scale_x, scale_w):
    m_per, k = x.shape
    n = w_mat.shape[1]
    n_per = n // N_DEV
    h_per = m_per // 2
    n_chunk = n_per // W_CHUNKS

    s = (scale_x * scale_w).astype(jnp.float32)

    def body(x_ref, w_hbm, s_ref, out_ref, cw_ref, ccw_ref, w8_ref,
             wstage_ref, acc_ref, cw_send, cw_recv, ccw_send, ccw_recv,
             copy_sems, wdma_sems):
        my_pos = lax.axis_index("i")
        left = (my_pos - 1) % N_DEV
        right = (my_pos + 1) % N_DEV

        barrier_sem = pltpu.get_barrier_semaphore()
        for nbr in [left, right]:
            pl.semaphore_signal(
                barrier_sem, inc=1,
                device_id=(nbr,), device_id_type=pl.DeviceIdType.MESH,
            )
        pl.semaphore_wait(barrier_sem, 2)

        scale = s_ref[0]

        cw_ref[0] = x_ref[:h_per, :].astype(jnp.float8_e5m2)
        ccw_ref[0] = x_ref[h_per:, :].astype(jnp.float8_e5m2)

        pending = []

        def compute_store(src, row0, sem_slot):
            acc_ref[sem_slot] = (
                jnp.dot(src, w8_ref[...], preferred_element_type=jnp.float32)
                * scale
            )
            cp = pltpu.make_async_copy(
                acc_ref.at[sem_slot],
                out_ref.at[pl.ds(row0, h_per), :],
                copy_sems.at[sem_slot],
            )
            cp.start()
            pending.append(cp)

        for hop in range(N_DEV - 1):
            s_slot = hop % 2
            r_slot = (hop + 1) % 2
            cw_rdma = pltpu.make_async_remote_copy(
                src_ref=cw_ref.at[s_slot],
                dst_ref=cw_ref.at[r_slot],
                send_sem=cw_send.at[s_slot],
                recv_sem=cw_recv.at[r_slot],
                device_id=(right,),
                device_id_type=pl.DeviceIdType.MESH,
            )
            ccw_rdma = pltpu.make_async_remote_copy(
                src_ref=ccw_ref.at[s_slot],
                dst_ref=ccw_ref.at[r_slot],
                send_sem=ccw_send.at[s_slot],
                recv_sem=ccw_recv.at[r_slot],
                device_id=(left,),
                device_id_type=pl.DeviceIdType.MESH,
            )
            cw_rdma.start()
            ccw_rdma.start()

            if hop == 0:
                dmas = []
                for c in range(W_CHUNKS):
                    d = pltpu.make_async_copy(
                        w_hbm.at[:, pl.ds((my_pos * n_per + c * n_chunk),
                                          n_chunk)],
                        wstage_ref.at[c % 2],
                        wdma_sems.at[c % 2],
                    )
                    d.start()
                    dmas.append(d)
                    if c >= 1:
                        dmas[c - 1].wait()
                        w8_ref[:, pl.ds((c - 1) * n_chunk, n_chunk)] = (
                            wstage_ref[(c - 1) % 2].astype(jnp.float8_e5m2)
                        )
                dmas[W_CHUNKS - 1].wait()
                w8_ref[:, pl.ds((W_CHUNKS - 1) * n_chunk, n_chunk)] = (
                    wstage_ref[(W_CHUNKS - 1) % 2].astype(jnp.float8_e5m2)
                )
                compute_store(cw_ref[0], my_pos * m_per, 0)
                compute_store(ccw_ref[0], my_pos * m_per + h_per, 1)
            else:
                while pending:
                    pending.pop(0).wait()
                o_cw = (my_pos - hop) % N_DEV
                o_ccw = (my_pos + hop) % N_DEV
                compute_store(cw_ref[s_slot], o_cw * m_per, 0)
                compute_store(ccw_ref[s_slot], o_ccw * m_per + h_per, 1)

            cw_rdma.wait()
            ccw_rdma.wait()

        while pending:
            pending.pop(0).wait()
        o_cw = (my_pos - (N_DEV - 1)) % N_DEV
        o_ccw = (my_pos + (N_DEV - 1)) % N_DEV
        compute_store(cw_ref[1], o_cw * m_per, 0)
        compute_store(ccw_ref[1], o_ccw * m_per + h_per, 1)
        while pending:
            pending.pop(0).wait()

    return pl.pallas_call(
        body,
        out_shape=jax.ShapeDtypeStruct((N_DEV * m_per, n_per), jnp.float32),
        in_specs=[
            pl.BlockSpec(memory_space=pltpu.VMEM),
            pl.BlockSpec(memory_space=pl.ANY),
            pl.BlockSpec(memory_space=pltpu.SMEM),
        ],
        out_specs=pl.BlockSpec(memory_space=pl.ANY),
        scratch_shapes=[
            pltpu.VMEM((2, h_per, k), jnp.float8_e5m2),
            pltpu.VMEM((2, h_per, k), jnp.float8_e5m2),
            pltpu.VMEM((k, n_per), jnp.float8_e5m2),
            pltpu.VMEM((2, k, n_chunk), jnp.float32),
            pltpu.VMEM((2, h_per, n_per), jnp.float32),
            pltpu.SemaphoreType.DMA((2,)),
            pltpu.SemaphoreType.DMA((2,)),
            pltpu.SemaphoreType.DMA((2,)),
            pltpu.SemaphoreType.DMA((2,)),
            pltpu.SemaphoreType.DMA((2,)),
            pltpu.SemaphoreType.DMA((2,)),
        ],
        compiler_params=pltpu.CompilerParams(
            collective_id=0, vmem_limit_bytes=60 * 1024 * 1024
        ),
    )(x, w_mat, s)
